# baseline (device time: 15662 ns/iter reference)
import jax
import jax.numpy as jnp
from jax import lax
from jax.experimental import pallas as pl
from jax.experimental.pallas import tpu as pltpu

BLK = 256


def kernel(x):
    m, n = x.shape
    nblk = m // BLK

    def body(x_ref, out_ref, acc_ref, comm_ref, send_sems, recv_sems):
        b = pl.program_id(0)
        my_x = lax.axis_index("x")
        my_y = lax.axis_index("y")
        peer = (my_x, 1 - my_y)

        @pl.when(b == 0)
        def _():
            barrier_sem = pltpu.get_barrier_semaphore()
            pl.semaphore_signal(
                barrier_sem, inc=1,
                device_id=peer, device_id_type=pl.DeviceIdType.MESH,
            )
            pl.semaphore_wait(barrier_sem, 1)

        rows = pl.ds(b * BLK, BLK)
        ones = jnp.ones((n, 1), jnp.float32)
        acc_ref[rows, :] = jax.lax.dot_general(
            x_ref[:, :], ones,
            dimension_numbers=(((1,), (0,)), ((), ())),
            preferred_element_type=jnp.float32,
        )
        rdma = pltpu.make_async_remote_copy(
            src_ref=acc_ref.at[rows],
            dst_ref=comm_ref.at[rows],
            send_sem=send_sems.at[b],
            recv_sem=recv_sems.at[b],
            device_id=peer,
            device_id_type=pl.DeviceIdType.MESH,
        )
        rdma.start()

        @pl.when(b == nblk - 1)
        def _():
            for h in range(nblk):
                hrows = pl.ds(h * BLK, BLK)
                drain = pltpu.make_async_remote_copy(
                    src_ref=acc_ref.at[hrows],
                    dst_ref=comm_ref.at[hrows],
                    send_sem=send_sems.at[h],
                    recv_sem=recv_sems.at[h],
                    device_id=peer,
                    device_id_type=pl.DeviceIdType.MESH,
                )
                drain.wait_send()
                drain.wait_recv()
            out_ref[:, :] = acc_ref[:, :] + comm_ref[:, :]

    return pl.pallas_call(
        body,
        grid=(nblk,),
        out_shape=jax.ShapeDtypeStruct((m, 1), jnp.float32),
        in_specs=[pl.BlockSpec((BLK, n), lambda b: (b, 0))],
        out_specs=pl.BlockSpec((m, 1), lambda b: (0, 0), memory_space=pltpu.VMEM),
        scratch_shapes=[
            pltpu.VMEM((m, 1), jnp.float32),
            pltpu.VMEM((m, 1), jnp.float32),
            pltpu.SemaphoreType.DMA((nblk,)),
            pltpu.SemaphoreType.DMA((nblk,)),
        ],
        compiler_params=pltpu.CompilerParams(collective_id=0),
    )(x)


# device time: 8881 ns/iter; 1.7635x vs baseline; 1.7635x over previous
import jax
import jax.numpy as jnp
from jax import lax
from jax.experimental import pallas as pl
from jax.experimental.pallas import tpu as pltpu

BLK = 256


def kernel(x):
    m, n = x.shape
    nblk = m // BLK

    def body(x_ref, out_ref, acc_ref, comm_ref, send_sems, recv_sems):
        b = pl.program_id(0)
        my_x = lax.axis_index("x")
        my_y = lax.axis_index("y")
        peer = (my_x, 1 - my_y)

        @pl.when(b == 0)
        def _():
            barrier_sem = pltpu.get_barrier_semaphore()
            pl.semaphore_signal(
                barrier_sem, inc=1,
                device_id=peer, device_id_type=pl.DeviceIdType.MESH,
            )
            pl.semaphore_wait(barrier_sem, 1)

        rows = pl.ds(b * BLK, BLK)
        acc_ref[rows] = jnp.sum(x_ref[:, :], axis=1)
        rdma = pltpu.make_async_remote_copy(
            src_ref=acc_ref.at[rows],
            dst_ref=comm_ref.at[rows],
            send_sem=send_sems.at[b],
            recv_sem=recv_sems.at[b],
            device_id=peer,
            device_id_type=pl.DeviceIdType.MESH,
        )
        rdma.start()

        @pl.when(b == nblk - 1)
        def _():
            for h in range(nblk):
                hrows = pl.ds(h * BLK, BLK)
                drain = pltpu.make_async_remote_copy(
                    src_ref=acc_ref.at[hrows],
                    dst_ref=comm_ref.at[hrows],
                    send_sem=send_sems.at[h],
                    recv_sem=recv_sems.at[h],
                    device_id=peer,
                    device_id_type=pl.DeviceIdType.MESH,
                )
                drain.wait_send()
                drain.wait_recv()
            out_ref[:, :] = (acc_ref[:] + comm_ref[:]).reshape(m, 1)

    return pl.pallas_call(
        body,
        grid=(nblk,),
        out_shape=jax.ShapeDtypeStruct((m, 1), jnp.float32),
        in_specs=[pl.BlockSpec((BLK, n), lambda b: (b, 0))],
        out_specs=pl.BlockSpec((m, 1), lambda b: (0, 0), memory_space=pltpu.VMEM),
        scratch_shapes=[
            pltpu.VMEM((m,), jnp.float32),
            pltpu.VMEM((m,), jnp.float32),
            pltpu.SemaphoreType.DMA((nblk,)),
            pltpu.SemaphoreType.DMA((nblk,)),
        ],
        compiler_params=pltpu.CompilerParams(collective_id=0),
    )(x)


# device time: 7959 ns/iter; 1.9678x vs baseline; 1.1158x over previous
import jax
import jax.numpy as jnp
from jax import lax
from jax.experimental import pallas as pl
from jax.experimental.pallas import tpu as pltpu


def kernel(x):
    m, n = x.shape

    def body(x_ref, out_ref, acc_ref, comm_ref, send_sem, recv_sem):
        my_x = lax.axis_index("x")
        my_y = lax.axis_index("y")
        peer = (my_x, 1 - my_y)

        barrier_sem = pltpu.get_barrier_semaphore()
        pl.semaphore_signal(
            barrier_sem, inc=1,
            device_id=peer, device_id_type=pl.DeviceIdType.MESH,
        )

        acc_ref[:] = jnp.sum(x_ref[:, :], axis=1)

        pl.semaphore_wait(barrier_sem, 1)

        rdma = pltpu.make_async_remote_copy(
            src_ref=acc_ref,
            dst_ref=comm_ref,
            send_sem=send_sem,
            recv_sem=recv_sem,
            device_id=peer,
            device_id_type=pl.DeviceIdType.MESH,
        )
        rdma.start()
        rdma.wait()

        out_ref[:, :] = (acc_ref[:] + comm_ref[:]).reshape(m, 1)

    return pl.pallas_call(
        body,
        out_shape=jax.ShapeDtypeStruct((m, 1), jnp.float32),
        in_specs=[pl.BlockSpec(memory_space=pltpu.VMEM)],
        out_specs=pl.BlockSpec(memory_space=pltpu.VMEM),
        scratch_shapes=[
            pltpu.VMEM((m,), jnp.float32),
            pltpu.VMEM((m,), jnp.float32),
            pltpu.SemaphoreType.DMA,
            pltpu.SemaphoreType.DMA,
        ],
        compiler_params=pltpu.CompilerParams(collective_id=0),
    )(x)
